# baseline (speedup 1.0000x reference)
import jax
import jax.numpy as jnp
from jax import lax
from jax.experimental import pallas as pl
from jax.experimental.pallas import tpu as pltpu
from jax.experimental.pallas import tpu_sc as plsc

N_CELLS = 8000
N_GENES = 2000
D = 128
E = 320000

NCP = 8192
NGP = 2048
CSHIFT = 13
GSHIFT = 11

NKEY = NGP * NCP
A_DTYPE = jnp.float32
KBLK = 1572864
NBLK = 11
DUMP = 2048
NSUB = 16
EP_T = E // NSUB
WIN = 128
NFULL = EP_T // WIN
TAIL = EP_T - NFULL * WIN
GSPAN = 1024
NSPAN = EP_T // GSPAN
SREM = EP_T - NSPAN * GSPAN

TILE_Z = (KBLK + DUMP) // NSUB
WB_CHUNK = 4096
WB_FULL = KBLK // NSUB
LAST_WB = (NKEY - (NBLK - 1) * KBLK) // NSUB

DEGC_PAD = 8192
DEGG_PAD = 2048
DEGC_DUMP = 8100
DEGG_DUMP = 2024

EW = E // 32
NFULL2 = EW // WIN
TAIL2 = EW - NFULL2 * WIN
NWIN2 = NFULL2 + 1
EW_PAD = NWIN2 * WIN

_sc_mesh = plsc.VectorSubcoreMesh(
    core_axis_name="c", subcore_axis_name="s", num_cores=2, num_subcores=NSUB)


def _build_graph_body(encc, encg, zeros_hbm, a_hbm, degc_hbm, degg_hbm,
                      accum, degc_s, degg_s,
                      cellb, gwin, idxb, valf, wb_t, sem, sem2):
    cid = lax.axis_index("c")
    sid = lax.axis_index("s")
    ebase = sid * EP_T

    pltpu.sync_copy(encc.at[pl.ds(ebase, EP_T)], cellb)

    @pl.loop(0, 512, step=16)
    def _zb(i):
        wb_t[0, pl.ds(i, 16)] = jnp.zeros((16,), jnp.float32)

    @pl.loop(0, WIN, step=16)
    def _vf(i):
        valf[pl.ds(i, 16)] = jnp.ones((16,), jnp.float32)

    iota16 = lax.iota(jnp.int32, 16)

    G = 6

    def pipelined_scatter(make_row, dest_s):
        def comp(g, half):
            for r in range(G):
                make_row(g * G + r, half * G + r)

        def fire_half(half):
            for r in range(G):
                pltpu.async_copy(valf, dest_s.at[idxb.at[half * G + r]], sem,
                                 add=True)

        def drain_g():
            for r in range(G):
                pltpu.make_async_copy(valf, dest_s.at[idxb.at[0]], sem).wait()

        comp(0, 0)
        fire_half(0)

        @pl.loop(1, NFULL // G)
        def _g(g):
            h = lax.bitwise_and(g, 1)
            comp(g, h)
            fire_half(h)
            drain_g()

        drain_g()

        for w in range(NFULL - NFULL % G, NFULL):
            make_row(w, 0)
            pltpu.sync_copy(valf, dest_s.at[idxb.at[0]], add=True)

    def deg_writeback(dest_s, dest_hbm, n_out):
        plsc.subcore_barrier()

        @pl.when(sid == 0)
        def _wb_deg():
            for h in range((n_out + WB_CHUNK - 1) // WB_CHUNK):
                size = min(WB_CHUNK, n_out - h * WB_CHUNK)
                pltpu.sync_copy(dest_s.at[pl.ds(h * WB_CHUNK, size)],
                                wb_t.at[1, pl.ds(0, size)])
                pltpu.sync_copy(wb_t.at[1, pl.ds(0, size)],
                                dest_hbm.at[pl.ds(h * WB_CHUNK, size)])

    @pl.when(cid == 0)
    def _deg_cells():
        pltpu.sync_copy(wb_t.at[0, pl.ds(0, 512)],
                        degc_s.at[pl.ds(sid * 512, 512)])
        plsc.subcore_barrier()

        def make_row(w, row):
            @pl.loop(0, WIN, step=16)
            def _chunk(j):
                idxb[row, pl.ds(j, 16)] = cellb[pl.ds(w * WIN + j, 16)]

        pipelined_scatter(make_row, degc_s)

        for j in range(0, TAIL, 16):
            idxb[0, pl.ds(j, 16)] = cellb[pl.ds(NFULL * WIN + j, 16)]
        for j in range(TAIL, WIN, 16):
            idxb[0, pl.ds(j, 16)] = DEGC_DUMP + iota16
        pltpu.sync_copy(valf, degc_s.at[idxb.at[0]], add=True)

        deg_writeback(degc_s, degc_hbm, DEGC_PAD)

    @pl.when(cid == 1)
    def _deg_genes():
        pltpu.sync_copy(wb_t.at[0, pl.ds(0, 128)],
                        degg_s.at[pl.ds(sid * 128, 128)])
        plsc.subcore_barrier()

        def span_rows(nwin):
            for r in range(nwin):
                @pl.loop(0, WIN, step=16)
                def _c(j):
                    idxb[r, pl.ds(j, 16)] = gwin[pl.ds(r * WIN + j, 16)]
                pltpu.async_copy(valf, degg_s.at[idxb.at[r]], sem, add=True)
            for r in range(nwin):
                pltpu.make_async_copy(valf, degg_s.at[idxb.at[0]], sem).wait()

        @pl.loop(0, NSPAN)
        def _s(s):
            pltpu.sync_copy(encg.at[pl.ds(ebase + s * GSPAN, GSPAN)], gwin)
            span_rows(8)

        pltpu.sync_copy(encg.at[pl.ds(ebase + NSPAN * GSPAN, SREM)],
                        gwin.at[pl.ds(0, SREM)])
        span_rows(SREM // WIN)
        for j in range(0, TAIL, 16):
            idxb[0, pl.ds(j, 16)] = gwin[pl.ds((SREM // WIN) * WIN + j, 16)]
        for j in range(TAIL, WIN, 16):
            idxb[0, pl.ds(j, 16)] = DEGG_DUMP + iota16
        pltpu.sync_copy(valf, degg_s.at[idxb.at[0]], add=True)

        deg_writeback(degg_s, degg_hbm, DEGG_PAD)

    def key_span(sbase, n):
        @pl.loop(0, n, step=16)
        def _c(j):
            p = sbase + j
            cellb[pl.ds(p, 16)] = lax.bitwise_or(
                lax.shift_left(gwin[pl.ds(j, 16)], CSHIFT), cellb[pl.ds(p, 16)])

    @pl.loop(0, NSPAN)
    def _ks(s):
        pltpu.sync_copy(encg.at[pl.ds(ebase + s * GSPAN, GSPAN)], gwin)
        key_span(s * GSPAN, GSPAN)

    pltpu.sync_copy(encg.at[pl.ds(ebase + NSPAN * GSPAN, SREM)],
                    gwin.at[pl.ds(0, SREM)])
    key_span(NSPAN * GSPAN, SREM)

    @pl.loop(0, 6)
    def _block(i):
        @pl.when((cid == 0) | (i < NBLK - 6))
        def _do():
            blk = cid * 6 + i
            base = blk * KBLK

            pltpu.sync_copy(zeros_hbm, accum.at[pl.ds(sid * TILE_Z, TILE_Z)])
            plsc.subcore_barrier()

            def one_chunk(w, row, j):
                k16 = cellb[pl.ds(w * WIN + j, 16)]
                local = k16 - base
                inb = plsc.bitcast(local, jnp.uint32) < jnp.uint32(KBLK)
                dump_idx = lax.bitwise_or(
                    jnp.int32(KBLK), lax.bitwise_and(local, DUMP - 1))
                idxb[row, pl.ds(j, 16)] = jnp.where(inb, local, dump_idx)

            def win_idx(w, row, nchunk=8):
                @pl.loop(0, nchunk * 16, step=32)
                def _chunk(j):
                    one_chunk(w, row, j)
                    one_chunk(w, row, j + 16)

            pipelined_scatter(win_idx, accum)

            win_idx(NFULL, 0, TAIL // 16)
            for j in range(TAIL, WIN, 16):
                idxb[0, pl.ds(j, 16)] = KBLK + j * 16 + iota16
            pltpu.sync_copy(valf, accum.at[idxb.at[0]], add=True)

            plsc.subcore_barrier()

            def _wb_pipe(span, nchunks):
                def hop1(h):
                    pltpu.sync_copy(
                        accum.at[pl.ds(sid * span + h * WB_CHUNK, WB_CHUNK)],
                        wb_t.at[lax.bitwise_and(h, 1)])

                def fire2(h):
                    pltpu.async_copy(
                        wb_t.at[lax.bitwise_and(h, 1)],
                        a_hbm.at[pl.ds(base + sid * span + h * WB_CHUNK,
                                       WB_CHUNK)], sem2)

                def drain1():
                    pltpu.make_async_copy(
                        wb_t.at[0], a_hbm.at[pl.ds(base, WB_CHUNK)],
                        sem2).wait()

                hop1(0)
                fire2(0)

                @pl.loop(1, nchunks)
                def _h(h):
                    hop1(h)
                    drain1()
                    fire2(h)

                drain1()

            @pl.when(blk < NBLK - 1)
            def _wb():
                _wb_pipe(WB_FULL, WB_FULL // WB_CHUNK)

            @pl.when(blk == NBLK - 1)
            def _wb_last():
                _wb_pipe(LAST_WB, LAST_WB // WB_CHUNK)

            plsc.subcore_barrier()


_build_graph = pl.kernel(
    _build_graph_body,
    out_type=(
        jax.ShapeDtypeStruct((NKEY,), A_DTYPE),
        jax.ShapeDtypeStruct((DEGC_PAD,), jnp.float32),
        jax.ShapeDtypeStruct((DEGG_PAD,), jnp.float32),
    ),
    mesh=_sc_mesh,
    scratch_types=[
        pltpu.VMEM_SHARED((KBLK + DUMP,), A_DTYPE),
        pltpu.VMEM_SHARED((DEGC_PAD,), jnp.float32),
        pltpu.VMEM_SHARED((DEGG_PAD,), jnp.float32),
        pltpu.VMEM((EP_T,), jnp.int32),
        pltpu.VMEM((GSPAN,), jnp.int32),
        pltpu.VMEM((2 * 6, WIN), jnp.int32),
        pltpu.VMEM((WIN,), jnp.float32),
        pltpu.VMEM((2, WB_CHUNK), jnp.float32),
        pltpu.SemaphoreType.DMA,
        pltpu.SemaphoreType.DMA,
    ],
)


GB = 128
NB_G = NGP // GB


def _conv_body(a_ref, degc_ref, degg_ref, cf_ref, gf_ref, ch_out, gh_out,
               cj_s, ci_s, xc_s, yg_s, cnext_s, gnext_s, ccur_s, gcur_s,
               ch_s, gh_s):
    l = pl.program_id(0)
    b = pl.program_id(1)

    @pl.when((l == 0) & (b == 0))
    def _init():
        cj_s[...] = lax.rsqrt(jnp.where(degc_ref[...] > 0.0, degc_ref[...], 1.0))
        ci_s[...] = lax.rsqrt(jnp.where(degg_ref[...] > 0.0, degg_ref[...], 1.0))
        ccur_s[...] = cf_ref[...]
        gcur_s[...] = gf_ref[...]
        ch_s[...] = cf_ref[...]
        gh_s[...] = gf_ref[...]

    @pl.when(b == 0)
    def _layer_start():
        xc_s[...] = (ccur_s[...] * cj_s[...]).astype(jnp.bfloat16)
        yg_s[...] = (gcur_s[...] * ci_s[...]).astype(jnp.bfloat16)
        cnext_s[...] = jnp.zeros_like(cnext_s)

    ab = a_ref[...].astype(jnp.bfloat16)
    gnew = jnp.dot(ab, xc_s[...], preferred_element_type=jnp.float32)
    gnew = gnew * ci_s[pl.ds(b * GB, GB), :]
    gnext_s[pl.ds(b * GB, GB), :] = gnew
    cnext_s[...] += lax.dot_general(
        ab, yg_s[pl.ds(b * GB, GB), :],
        dimension_numbers=(((0,), (0,)), ((), ())),
        preferred_element_type=jnp.float32)

    @pl.when(b == NB_G - 1)
    def _layer_end():
        cnew = cnext_s[...] * cj_s[...]
        ch_s[...] += 0.5 * cnew
        gh_s[...] += 0.5 * gnext_s[...]
        ccur_s[...] = cnew
        gcur_s[...] = gnext_s[...]

    @pl.when((l == 1) & (b == NB_G - 1))
    def _finish():
        ch_out[...] = ch_s[...]
        gh_out[...] = gh_s[...]


def _run_conv(a2d, degc, degg, cf, gf):
    return pl.pallas_call(
        _conv_body,
        grid=(2, NB_G),
        in_specs=[
            pl.BlockSpec((GB, NCP), lambda l, b: (b, 0)),
            pl.BlockSpec((NCP, 1), lambda l, b: (0, 0)),
            pl.BlockSpec((NGP, 1), lambda l, b: (0, 0)),
            pl.BlockSpec((NCP, D), lambda l, b: (0, 0)),
            pl.BlockSpec((NGP, D), lambda l, b: (0, 0)),
        ],
        out_specs=[
            pl.BlockSpec((NCP, D), lambda l, b: (0, 0)),
            pl.BlockSpec((NGP, D), lambda l, b: (0, 0)),
        ],
        out_shape=[
            jax.ShapeDtypeStruct((NCP, D), jnp.float32),
            jax.ShapeDtypeStruct((NGP, D), jnp.float32),
        ],
        scratch_shapes=[
            pltpu.VMEM((NCP, 1), jnp.float32),
            pltpu.VMEM((NGP, 1), jnp.float32),
            pltpu.VMEM((NCP, D), jnp.bfloat16),
            pltpu.VMEM((NGP, D), jnp.bfloat16),
            pltpu.VMEM((NCP, D), jnp.float32),
            pltpu.VMEM((NGP, D), jnp.float32),
            pltpu.VMEM((NCP, D), jnp.float32),
            pltpu.VMEM((NGP, D), jnp.float32),
            pltpu.VMEM((NCP, D), jnp.float32),
            pltpu.VMEM((NGP, D), jnp.float32),
        ],
    )(a2d, degc, degg, cf, gf)


SB = 1024


def _score_body(ch_ref, gh_ref, s_ref):
    s_ref[...] = lax.dot_general(
        ch_ref[...].astype(jnp.bfloat16), gh_ref[...].astype(jnp.bfloat16),
        dimension_numbers=(((1,), (1,)), ((), ())),
        preferred_element_type=jnp.float32)


def _run_score(ch, gh):
    return pl.pallas_call(
        _score_body,
        grid=(NCP // SB,),
        in_specs=[
            pl.BlockSpec((SB, D), lambda b: (b, 0)),
            pl.BlockSpec((NGP, D), lambda b: (0, 0)),
        ],
        out_specs=pl.BlockSpec((SB, NGP), lambda b: (b, 0)),
        out_shape=jax.ShapeDtypeStruct((NCP, NGP), jnp.float32),
    )(ch, gh)


def _decode_body(sflat, pc, pg, nc, ng, pos_out, neg_out,
                 ib, jb, keyb, valb, sem):
    cid = lax.axis_index("c")
    sid = lax.axis_index("s")
    wid = sid * 2 + cid
    base = wid * EW

    def load_and_key(cells_hbm, genes_hbm, wbase):
        pltpu.sync_copy(cells_hbm.at[pl.ds(base, EW)], ib)
        pltpu.sync_copy(genes_hbm.at[pl.ds(base, EW)], jb)

        @pl.loop(0, NFULL2)
        def _keys(w):
            @pl.loop(0, WIN, step=16)
            def _chunk(j):
                p = w * WIN + j
                keyb[wbase + w, pl.ds(j, 16)] = lax.bitwise_or(
                    lax.shift_left(ib[pl.ds(p, 16)], GSHIFT), jb[pl.ds(p, 16)])

        for j in range(0, TAIL2, 16):
            p = NFULL2 * WIN + j
            keyb[wbase + NFULL2, pl.ds(j, 16)] = lax.bitwise_or(
                lax.shift_left(ib[pl.ds(p, 16)], GSHIFT), jb[pl.ds(p, 16)])
        for j in range(TAIL2, WIN, 16):
            keyb[wbase + NFULL2, pl.ds(j, 16)] = jnp.zeros((16,), jnp.int32)

    load_and_key(pc, pg, 0)
    load_and_key(nc, ng, NWIN2)

    NW_ALL = 2 * NWIN2
    GW = 16

    def fire(w):
        pltpu.async_copy(sflat.at[keyb.at[w]],
                         valb.at[pl.ds(w * WIN, WIN)], sem)

    def drain(n):
        for _ in range(n):
            pltpu.make_async_copy(sflat.at[keyb.at[0]],
                                  valb.at[pl.ds(0, WIN)], sem).wait()

    for r in range(GW):
        fire(r)

    @pl.loop(1, NW_ALL // GW)
    def _g(g):
        for r in range(GW):
            fire(g * GW + r)
        drain(GW)

    for w in range(NW_ALL - NW_ALL % GW, NW_ALL):
        fire(w)
    drain(GW + NW_ALL % GW)

    pltpu.sync_copy(valb.at[pl.ds(0, EW)], pos_out.at[pl.ds(base, EW)])
    pltpu.sync_copy(valb.at[pl.ds(NWIN2 * WIN, EW)], neg_out.at[pl.ds(base, EW)])


_decode = pl.kernel(
    _decode_body,
    out_type=(
        jax.ShapeDtypeStruct((E,), jnp.float32),
        jax.ShapeDtypeStruct((E,), jnp.float32),
    ),
    mesh=_sc_mesh,
    scratch_types=[
        pltpu.VMEM((EW,), jnp.int32),
        pltpu.VMEM((EW,), jnp.int32),
        pltpu.VMEM((2 * NWIN2, WIN), jnp.int32),
        pltpu.VMEM((2 * EW_PAD,), jnp.float32),
        pltpu.SemaphoreType.DMA,
    ],
)


def kernel(cell_feature, gene_feature, enc_cell, enc_gene,
           pos_cell, pos_gene, neg_cell, neg_gene):
    zeros_hbm = jnp.zeros((TILE_Z,), A_DTYPE)
    a_flat, deg_c, deg_g = _build_graph(enc_cell, enc_gene, zeros_hbm)
    a2d = a_flat.reshape(NGP, NCP)
    cfp = jnp.pad(cell_feature, ((0, NCP - N_CELLS), (0, 0)))
    gfp = jnp.pad(gene_feature, ((0, NGP - N_GENES), (0, 0)))
    ch, gh = _run_conv(a2d, deg_c.reshape(NCP, 1), deg_g.reshape(NGP, 1),
                       cfp, gfp)
    s = _run_score(ch, gh)
    pos_pre, neg_pre = _decode(s.reshape(NKEY), pos_cell, pos_gene,
                               neg_cell, neg_gene)
    return (pos_pre, neg_pre)


# speedup vs baseline: 1.0129x; 1.0129x over previous
import jax
import jax.numpy as jnp
from jax import lax
from jax.experimental import pallas as pl
from jax.experimental.pallas import tpu as pltpu
from jax.experimental.pallas import tpu_sc as plsc

N_CELLS = 8000
N_GENES = 2000
D = 128
E = 320000

NCP = 8192
NGP = 2048
CSHIFT = 13
GSHIFT = 11

NKEY = NGP * NCP
A_DTYPE = jnp.float32
KBLK = 1572864
NBLK = 11
DUMP = 2048
NSUB = 16
EP_T = E // NSUB
WIN = 128
NFULL = EP_T // WIN
TAIL = EP_T - NFULL * WIN
GSPAN = 1024
NSPAN = EP_T // GSPAN
SREM = EP_T - NSPAN * GSPAN

TILE_Z = (KBLK + DUMP) // NSUB
WB_CHUNK = 8192
WB_FULL = KBLK // NSUB
LAST_WB = (NKEY - (NBLK - 1) * KBLK) // NSUB

DEGC_PAD = 8192
DEGG_PAD = 2048
DEGC_DUMP = 8100
DEGG_DUMP = 2024

EW = E // 32
NFULL2 = EW // WIN
TAIL2 = EW - NFULL2 * WIN
NWIN2 = NFULL2 + 1
EW_PAD = NWIN2 * WIN

_sc_mesh = plsc.VectorSubcoreMesh(
    core_axis_name="c", subcore_axis_name="s", num_cores=2, num_subcores=NSUB)


def _build_graph_body(encc, encg, zeros_hbm, a_hbm, degc_hbm, degg_hbm,
                      accum, degc_s, degg_s,
                      cellb, gwin, idxb, valf, wb_t, sem):
    cid = lax.axis_index("c")
    sid = lax.axis_index("s")
    ebase = sid * EP_T

    pltpu.sync_copy(encc.at[pl.ds(ebase, EP_T)], cellb)

    @pl.loop(0, 512, step=16)
    def _zb(i):
        wb_t[pl.ds(i, 16)] = jnp.zeros((16,), jnp.float32)

    @pl.loop(0, WIN, step=16)
    def _vf(i):
        valf[pl.ds(i, 16)] = jnp.ones((16,), jnp.float32)

    iota16 = lax.iota(jnp.int32, 16)

    G = 6

    def pipelined_scatter(make_row, dest_s):
        def comp(g, half):
            for r in range(G):
                make_row(g * G + r, half * G + r)

        def fire_half(half):
            for r in range(G):
                pltpu.async_copy(valf, dest_s.at[idxb.at[half * G + r]], sem,
                                 add=True)

        def drain_g():
            for r in range(G):
                pltpu.make_async_copy(valf, dest_s.at[idxb.at[0]], sem).wait()

        comp(0, 0)
        fire_half(0)

        @pl.loop(1, NFULL // G)
        def _g(g):
            h = lax.bitwise_and(g, 1)
            comp(g, h)
            fire_half(h)
            drain_g()

        drain_g()

        for w in range(NFULL - NFULL % G, NFULL):
            make_row(w, 0)
            pltpu.sync_copy(valf, dest_s.at[idxb.at[0]], add=True)

    def deg_writeback(dest_s, dest_hbm, n_out):
        plsc.subcore_barrier()

        @pl.when(sid == 0)
        def _wb_deg():
            pltpu.sync_copy(dest_s.at[pl.ds(0, n_out)], wb_t.at[pl.ds(0, n_out)])
            pltpu.sync_copy(wb_t.at[pl.ds(0, n_out)], dest_hbm)

    @pl.when(cid == 0)
    def _deg_cells():
        pltpu.sync_copy(wb_t.at[pl.ds(0, 512)],
                        degc_s.at[pl.ds(sid * 512, 512)])
        plsc.subcore_barrier()

        def make_row(w, row):
            @pl.loop(0, WIN, step=16)
            def _chunk(j):
                idxb[row, pl.ds(j, 16)] = cellb[pl.ds(w * WIN + j, 16)]

        pipelined_scatter(make_row, degc_s)

        for j in range(0, TAIL, 16):
            idxb[0, pl.ds(j, 16)] = cellb[pl.ds(NFULL * WIN + j, 16)]
        for j in range(TAIL, WIN, 16):
            idxb[0, pl.ds(j, 16)] = DEGC_DUMP + iota16
        pltpu.sync_copy(valf, degc_s.at[idxb.at[0]], add=True)

        deg_writeback(degc_s, degc_hbm, DEGC_PAD)

    @pl.when(cid == 1)
    def _deg_genes():
        pltpu.sync_copy(wb_t.at[pl.ds(0, 128)],
                        degg_s.at[pl.ds(sid * 128, 128)])
        plsc.subcore_barrier()

        def span_rows(nwin):
            for r in range(nwin):
                @pl.loop(0, WIN, step=16)
                def _c(j):
                    idxb[r, pl.ds(j, 16)] = gwin[pl.ds(r * WIN + j, 16)]
                pltpu.async_copy(valf, degg_s.at[idxb.at[r]], sem, add=True)
            for r in range(nwin):
                pltpu.make_async_copy(valf, degg_s.at[idxb.at[0]], sem).wait()

        @pl.loop(0, NSPAN)
        def _s(s):
            pltpu.sync_copy(encg.at[pl.ds(ebase + s * GSPAN, GSPAN)], gwin)
            span_rows(8)

        pltpu.sync_copy(encg.at[pl.ds(ebase + NSPAN * GSPAN, SREM)],
                        gwin.at[pl.ds(0, SREM)])
        span_rows(SREM // WIN)
        for j in range(0, TAIL, 16):
            idxb[0, pl.ds(j, 16)] = gwin[pl.ds((SREM // WIN) * WIN + j, 16)]
        for j in range(TAIL, WIN, 16):
            idxb[0, pl.ds(j, 16)] = DEGG_DUMP + iota16
        pltpu.sync_copy(valf, degg_s.at[idxb.at[0]], add=True)

        deg_writeback(degg_s, degg_hbm, DEGG_PAD)

    def key_span(sbase, n):
        @pl.loop(0, n, step=16)
        def _c(j):
            p = sbase + j
            cellb[pl.ds(p, 16)] = lax.bitwise_or(
                lax.shift_left(gwin[pl.ds(j, 16)], CSHIFT), cellb[pl.ds(p, 16)])

    @pl.loop(0, NSPAN)
    def _ks(s):
        pltpu.sync_copy(encg.at[pl.ds(ebase + s * GSPAN, GSPAN)], gwin)
        key_span(s * GSPAN, GSPAN)

    pltpu.sync_copy(encg.at[pl.ds(ebase + NSPAN * GSPAN, SREM)],
                    gwin.at[pl.ds(0, SREM)])
    key_span(NSPAN * GSPAN, SREM)

    @pl.loop(0, 6)
    def _block(i):
        @pl.when((cid == 0) | (i < NBLK - 6))
        def _do():
            blk = cid * 6 + i
            base = blk * KBLK

            pltpu.sync_copy(zeros_hbm, accum.at[pl.ds(sid * TILE_Z, TILE_Z)])
            plsc.subcore_barrier()

            def one_chunk(w, row, j):
                k16 = cellb[pl.ds(w * WIN + j, 16)]
                local = k16 - base
                inb = plsc.bitcast(local, jnp.uint32) < jnp.uint32(KBLK)
                dump_idx = lax.bitwise_or(
                    jnp.int32(KBLK), lax.bitwise_and(local, DUMP - 1))
                idxb[row, pl.ds(j, 16)] = jnp.where(inb, local, dump_idx)

            def win_idx(w, row, nchunk=8):
                @pl.loop(0, nchunk * 16, step=32)
                def _chunk(j):
                    one_chunk(w, row, j)
                    one_chunk(w, row, j + 16)

            pipelined_scatter(win_idx, accum)

            win_idx(NFULL, 0, TAIL // 16)
            for j in range(TAIL, WIN, 16):
                idxb[0, pl.ds(j, 16)] = KBLK + j * 16 + iota16
            pltpu.sync_copy(valf, accum.at[idxb.at[0]], add=True)

            plsc.subcore_barrier()

            def _two_hop(off):
                pltpu.sync_copy(accum.at[pl.ds(off, WB_CHUNK)], wb_t)
                pltpu.sync_copy(wb_t, a_hbm.at[pl.ds(base + off, WB_CHUNK)])

            @pl.when(blk < NBLK - 1)
            def _wb():
                @pl.loop(0, WB_FULL // WB_CHUNK)
                def _part(h):
                    _two_hop(sid * WB_FULL + h * WB_CHUNK)

            @pl.when(blk == NBLK - 1)
            def _wb_last():
                @pl.loop(0, LAST_WB // WB_CHUNK)
                def _part(h):
                    _two_hop(sid * LAST_WB + h * WB_CHUNK)

            plsc.subcore_barrier()


_build_graph = pl.kernel(
    _build_graph_body,
    out_type=(
        jax.ShapeDtypeStruct((NKEY,), A_DTYPE),
        jax.ShapeDtypeStruct((DEGC_PAD,), jnp.float32),
        jax.ShapeDtypeStruct((DEGG_PAD,), jnp.float32),
    ),
    mesh=_sc_mesh,
    scratch_types=[
        pltpu.VMEM_SHARED((KBLK + DUMP,), A_DTYPE),
        pltpu.VMEM_SHARED((DEGC_PAD,), jnp.float32),
        pltpu.VMEM_SHARED((DEGG_PAD,), jnp.float32),
        pltpu.VMEM((EP_T,), jnp.int32),
        pltpu.VMEM((GSPAN,), jnp.int32),
        pltpu.VMEM((2 * 6, WIN), jnp.int32),
        pltpu.VMEM((WIN,), jnp.float32),
        pltpu.VMEM((WB_CHUNK,), jnp.float32),
        pltpu.SemaphoreType.DMA,
    ],
)


GB = 128
NB_G = NGP // GB


def _conv_body(a_ref, degc_ref, degg_ref, cf_ref, gf_ref, ch_out, gh_out,
               cj_s, ci_s, xc_s, yg_s, cnext_s, gnext_s, ccur_s, gcur_s,
               ch_s, gh_s):
    l = pl.program_id(0)
    b = pl.program_id(1)

    @pl.when((l == 0) & (b == 0))
    def _init():
        cj_s[...] = lax.rsqrt(jnp.where(degc_ref[...] > 0.0, degc_ref[...], 1.0))
        ci_s[...] = lax.rsqrt(jnp.where(degg_ref[...] > 0.0, degg_ref[...], 1.0))
        ccur_s[...] = cf_ref[...]
        gcur_s[...] = gf_ref[...]
        ch_s[...] = cf_ref[...]
        gh_s[...] = gf_ref[...]

    @pl.when(b == 0)
    def _layer_start():
        xc_s[...] = (ccur_s[...] * cj_s[...]).astype(jnp.bfloat16)
        yg_s[...] = (gcur_s[...] * ci_s[...]).astype(jnp.bfloat16)
        cnext_s[...] = jnp.zeros_like(cnext_s)

    ab = a_ref[...].astype(jnp.bfloat16)
    gnew = jnp.dot(ab, xc_s[...], preferred_element_type=jnp.float32)
    gnew = gnew * ci_s[pl.ds(b * GB, GB), :]
    gnext_s[pl.ds(b * GB, GB), :] = gnew
    cnext_s[...] += lax.dot_general(
        ab, yg_s[pl.ds(b * GB, GB), :],
        dimension_numbers=(((0,), (0,)), ((), ())),
        preferred_element_type=jnp.float32)

    @pl.when(b == NB_G - 1)
    def _layer_end():
        cnew = cnext_s[...] * cj_s[...]
        ch_s[...] += 0.5 * cnew
        gh_s[...] += 0.5 * gnext_s[...]
        ccur_s[...] = cnew
        gcur_s[...] = gnext_s[...]

    @pl.when((l == 1) & (b == NB_G - 1))
    def _finish():
        ch_out[...] = ch_s[...]
        gh_out[...] = gh_s[...]


def _run_conv(a2d, degc, degg, cf, gf):
    return pl.pallas_call(
        _conv_body,
        grid=(2, NB_G),
        in_specs=[
            pl.BlockSpec((GB, NCP), lambda l, b: (b, 0)),
            pl.BlockSpec((NCP, 1), lambda l, b: (0, 0)),
            pl.BlockSpec((NGP, 1), lambda l, b: (0, 0)),
            pl.BlockSpec((NCP, D), lambda l, b: (0, 0)),
            pl.BlockSpec((NGP, D), lambda l, b: (0, 0)),
        ],
        out_specs=[
            pl.BlockSpec((NCP, D), lambda l, b: (0, 0)),
            pl.BlockSpec((NGP, D), lambda l, b: (0, 0)),
        ],
        out_shape=[
            jax.ShapeDtypeStruct((NCP, D), jnp.float32),
            jax.ShapeDtypeStruct((NGP, D), jnp.float32),
        ],
        scratch_shapes=[
            pltpu.VMEM((NCP, 1), jnp.float32),
            pltpu.VMEM((NGP, 1), jnp.float32),
            pltpu.VMEM((NCP, D), jnp.bfloat16),
            pltpu.VMEM((NGP, D), jnp.bfloat16),
            pltpu.VMEM((NCP, D), jnp.float32),
            pltpu.VMEM((NGP, D), jnp.float32),
            pltpu.VMEM((NCP, D), jnp.float32),
            pltpu.VMEM((NGP, D), jnp.float32),
            pltpu.VMEM((NCP, D), jnp.float32),
            pltpu.VMEM((NGP, D), jnp.float32),
        ],
    )(a2d, degc, degg, cf, gf)


SB = 1024


def _score_body(ch_ref, gh_ref, s_ref):
    s_ref[...] = lax.dot_general(
        ch_ref[...].astype(jnp.bfloat16), gh_ref[...].astype(jnp.bfloat16),
        dimension_numbers=(((1,), (1,)), ((), ())),
        preferred_element_type=jnp.float32)


def _run_score(ch, gh):
    return pl.pallas_call(
        _score_body,
        grid=(NCP // SB,),
        in_specs=[
            pl.BlockSpec((SB, D), lambda b: (b, 0)),
            pl.BlockSpec((NGP, D), lambda b: (0, 0)),
        ],
        out_specs=pl.BlockSpec((SB, NGP), lambda b: (b, 0)),
        out_shape=jax.ShapeDtypeStruct((NCP, NGP), jnp.float32),
    )(ch, gh)


def _decode_body(sflat, pc, pg, nc, ng, pos_out, neg_out,
                 ib, jb, keyb, valb, sem):
    cid = lax.axis_index("c")
    sid = lax.axis_index("s")
    wid = sid * 2 + cid
    base = wid * EW

    def load_and_key(cells_hbm, genes_hbm, wbase):
        pltpu.sync_copy(cells_hbm.at[pl.ds(base, EW)], ib)
        pltpu.sync_copy(genes_hbm.at[pl.ds(base, EW)], jb)

        @pl.loop(0, NFULL2)
        def _keys(w):
            @pl.loop(0, WIN, step=16)
            def _chunk(j):
                p = w * WIN + j
                keyb[wbase + w, pl.ds(j, 16)] = lax.bitwise_or(
                    lax.shift_left(ib[pl.ds(p, 16)], GSHIFT), jb[pl.ds(p, 16)])

        for j in range(0, TAIL2, 16):
            p = NFULL2 * WIN + j
            keyb[wbase + NFULL2, pl.ds(j, 16)] = lax.bitwise_or(
                lax.shift_left(ib[pl.ds(p, 16)], GSHIFT), jb[pl.ds(p, 16)])
        for j in range(TAIL2, WIN, 16):
            keyb[wbase + NFULL2, pl.ds(j, 16)] = jnp.zeros((16,), jnp.int32)

    load_and_key(pc, pg, 0)
    load_and_key(nc, ng, NWIN2)

    NW_ALL = 2 * NWIN2
    GW = 16

    def fire(w):
        pltpu.async_copy(sflat.at[keyb.at[w]],
                         valb.at[pl.ds(w * WIN, WIN)], sem)

    def drain(n):
        for _ in range(n):
            pltpu.make_async_copy(sflat.at[keyb.at[0]],
                                  valb.at[pl.ds(0, WIN)], sem).wait()

    for r in range(GW):
        fire(r)

    @pl.loop(1, NW_ALL // GW)
    def _g(g):
        for r in range(GW):
            fire(g * GW + r)
        drain(GW)

    for w in range(NW_ALL - NW_ALL % GW, NW_ALL):
        fire(w)
    drain(GW + NW_ALL % GW)

    pltpu.sync_copy(valb.at[pl.ds(0, EW)], pos_out.at[pl.ds(base, EW)])
    pltpu.sync_copy(valb.at[pl.ds(NWIN2 * WIN, EW)], neg_out.at[pl.ds(base, EW)])


_decode = pl.kernel(
    _decode_body,
    out_type=(
        jax.ShapeDtypeStruct((E,), jnp.float32),
        jax.ShapeDtypeStruct((E,), jnp.float32),
    ),
    mesh=_sc_mesh,
    scratch_types=[
        pltpu.VMEM((EW,), jnp.int32),
        pltpu.VMEM((EW,), jnp.int32),
        pltpu.VMEM((2 * NWIN2, WIN), jnp.int32),
        pltpu.VMEM((2 * EW_PAD,), jnp.float32),
        pltpu.SemaphoreType.DMA,
    ],
)


def kernel(cell_feature, gene_feature, enc_cell, enc_gene,
           pos_cell, pos_gene, neg_cell, neg_gene):
    zeros_hbm = jnp.zeros((TILE_Z,), A_DTYPE)
    a_flat, deg_c, deg_g = _build_graph(enc_cell, enc_gene, zeros_hbm)
    a2d = a_flat.reshape(NGP, NCP)
    cfp = jnp.pad(cell_feature, ((0, NCP - N_CELLS), (0, 0)))
    gfp = jnp.pad(gene_feature, ((0, NGP - N_GENES), (0, 0)))
    ch, gh = _run_conv(a2d, deg_c.reshape(NCP, 1), deg_g.reshape(NGP, 1),
                       cfp, gfp)
    s = _run_score(ch, gh)
    pos_pre, neg_pre = _decode(s.reshape(NKEY), pos_cell, pos_gene,
                               neg_cell, neg_gene)
    return (pos_pre, neg_pre)
